# baseline (speedup 1.0000x reference)
import functools

import jax
import jax.numpy as jnp
from jax import lax
from jax.experimental import pallas as pl
from jax.experimental.pallas import tpu as pltpu
from jax.experimental.pallas import tpu_sc as plsc

N_NODES = 10000
E = 160000
R0 = 5.0

PW = 128
NC = 2
NS = 16
CHUNK = 128
GRP = 4
NSPLIT = 4
CPT = 80 // NSPLIT
EPT = CPT * CHUNK
EPH = NS * EPT
EP = NSPLIT * EPH
RPT = 632
NPAD = NS * RPT
TCB = 8192


def _leaky(x):
    return jnp.where(x >= 0, x, 0.1 * x)


def _dot(a, b):
    return lax.dot(a, b, precision=lax.Precision.HIGHEST,
                   preferred_element_type=jnp.float32)


def _edge_phi_body(r_ref, w0, b0, wd, w1, b1, w2, b2, w3, b3, wv, wdd,
                   out_ref):
    rt = r_ref[...]
    d2 = jnp.sum(rt * rt, axis=0, keepdims=True)
    x = jnp.sqrt(d2 + 1e-12) * (1.0 / R0)
    mu = lax.broadcasted_iota(jnp.int32, (8, 1), 0).astype(jnp.float32) / 7.0
    enc = jnp.exp(-0.5 * ((x - mu) * 8.0) ** 2)
    h = _dot(w0[...], enc) + b0[...]
    direct = _dot(wd[...], h)
    y = _leaky(_dot(w1[...], h) + b1[...])
    y = _leaky(_dot(w2[...], y) + b2[...])
    y = _dot(w3[...], y) + b3[...]
    rad = direct + y
    gv = _dot(wv[...], rad)
    gd = _dot(wdd[...], rad)
    rs = rt * (7.0 / R0)
    n2 = jnp.sum(rs * rs, axis=0, keepdims=True)
    rh = rs / jnp.sqrt(1.0 + n2)
    r0_, r1_, r2_ = rh[0:1], rh[1:2], rh[2:3]
    p0 = jnp.concatenate(
        [rad, gv * r0_, gv * r1_, gv * r2_, gd * (r0_ * r0_)], axis=0)
    p1 = jnp.concatenate(
        [gd * (r0_ * r1_), gd * (r0_ * r2_), gd * (r1_ * r1_),
         gd * (r1_ * r2_), gd * (r2_ * r2_)], axis=0)
    zpad = jnp.zeros((PW - 80, p0.shape[1]), jnp.float32)
    out_ref[0] = jnp.concatenate([p0, zpad], axis=0).T
    out_ref[1] = jnp.concatenate([p1, zpad], axis=0).T


def _edge_payload(r_t, *ws):
    wspecs = [pl.BlockSpec(w.shape, lambda i: (0, 0)) for w in ws]
    return pl.pallas_call(
        _edge_phi_body,
        grid=(EPH // TCB,),
        in_specs=[pl.BlockSpec((3, TCB), lambda i: (0, i))] + wspecs,
        out_specs=pl.BlockSpec((2, TCB, PW), lambda i: (0, i, 0)),
        out_shape=jax.ShapeDtypeStruct((2, EPH, PW), jnp.float32),
    )(r_t, *ws)


def _unpack_body(acc_ref, aa_ref, v_ref, d_ref):
    o0 = acc_ref[0]
    o1 = acc_ref[1]
    aa_ref[...] = o0[:, 0:16]
    j = lax.broadcasted_iota(jnp.int32, (48, 48), 0)
    k = lax.broadcasted_iota(jnp.int32, (48, 48), 1)
    pv = ((j % 16) * 3 + j // 16 == k).astype(jnp.float32)
    v_ref[...] = _dot(o0[:, 16:64], pv)
    d6 = jnp.concatenate([o0[:, 64:80], o1], axis=1)
    j = lax.broadcasted_iota(jnp.int32, (96, 144), 0)
    k = lax.broadcasted_iota(jnp.int32, (96, 144), 1)
    r = (k % 9) // 3
    s = k % 3
    lo = jnp.minimum(r, s)
    hi = jnp.maximum(r, s)
    pair = lo * 3 - (lo * (lo - 1)) // 2 + (hi - lo)
    pd = ((j % 16 == k // 9) & (j // 16 == pair)).astype(jnp.float32)
    d_ref[...] = _dot(d6, pd)


def _unpack(acc):
    nb = NPAD // RPT
    return pl.pallas_call(
        _unpack_body,
        grid=(nb,),
        in_specs=[pl.BlockSpec((2, RPT, 80), lambda i: (0, i, 0))],
        out_specs=[
            pl.BlockSpec((RPT, 16), lambda i: (i, 0)),
            pl.BlockSpec((RPT, 48), lambda i: (i, 0)),
            pl.BlockSpec((RPT, 144), lambda i: (i, 0)),
        ],
        out_shape=[
            jax.ShapeDtypeStruct((NPAD, 16), jnp.float32),
            jax.ShapeDtypeStruct((NPAD, 48), jnp.float32),
            jax.ShapeDtypeStruct((NPAD, 144), jnp.float32),
        ],
    )(acc)


@functools.cache
def _build_segment_sum_sc():
    mesh = plsc.VectorSubcoreMesh(
        core_axis_name="c", subcore_axis_name="s",
        num_cores=NC, num_subcores=NS)
    return pl.kernel(
        _segment_sum_sc_body,
        out_type=jax.ShapeDtypeStruct((NC, NPAD, 80), jnp.float32),
        mesh=mesh,
        scratch_types=[
            pltpu.VMEM_SHARED((NPAD, 80), jnp.float32),
            pltpu.VMEM((GRP * CHUNK, 80), jnp.float32),
            pltpu.VMEM((GRP, CHUNK), jnp.int32),
            pltpu.VMEM((64, 80), jnp.float32),
            pltpu.SemaphoreType.DMA,
        ],
        compiler_params=pltpu.CompilerParams(use_tc_tiling_on_sc=False),
    )


def _segment_sum_sc_body(pay_hbm, idx_hbm, out_hbm, acc, pbuf, idxbuf, zbuf,
                         sem):
    c = lax.axis_index("c")
    s = lax.axis_index("s")

    def zrow(r, carry):
        for l in range(80 // 16):
            zbuf[r, pl.ds(l * 16, 16)] = jnp.zeros((16,), jnp.float32)
        return carry

    lax.fori_loop(0, 64, zrow, 0)
    for z in range(9):
        pltpu.sync_copy(zbuf, acc.at[pl.ds(s * RPT + z * 64, 64)])
    pltpu.sync_copy(zbuf.at[pl.ds(0, 56)],
                    acc.at[pl.ds(s * RPT + 576, 56)])
    plsc.subcore_barrier()

    def body(g, carry):
        base = s * EPT + g * (GRP * CHUNK)
        for k in range(GRP):
            pltpu.sync_copy(idx_hbm.at[pl.ds(base + k * CHUNK, CHUNK)],
                            idxbuf.at[k])
        pltpu.sync_copy(
            pay_hbm.at[c, pl.ds(base, GRP * CHUNK), pl.ds(0, 80)], pbuf)
        descs = [
            pltpu.async_copy(
                pbuf.at[pl.ds(k * CHUNK, CHUNK)],
                acc.at[idxbuf.at[k]],
                sem, add=True)
            for k in range(GRP)
        ]
        for dsc in descs:
            dsc.wait()
        return carry

    lax.fori_loop(0, CPT // GRP, body, 0)
    plsc.subcore_barrier()
    pltpu.sync_copy(acc.at[pl.ds(s * RPT, RPT)],
                    out_hbm.at[c, pl.ds(s * RPT, RPT)])


def kernel(r_ij, edge_index, W0, b0, Wd, W1, b1, W2, b2, W3, b3, Wv, Wdd):
    src = edge_index[0].astype(jnp.int32)
    r_t = jnp.zeros((3, EP), jnp.float32).at[:, :E].set(r_ij.T)
    idx = jnp.full((EP,), N_NODES, jnp.int32).at[:E].set(src)
    ws = (W0, b0.reshape(-1, 1), Wd, W1, b1.reshape(-1, 1), W2,
          b2.reshape(-1, 1), W3, b3.reshape(-1, 1), Wv, Wdd)
    seg = _build_segment_sum_sc()
    accs = []
    for h in range(NSPLIT):
        pay = _edge_payload(r_t[:, h * EPH:(h + 1) * EPH], *ws)
        accs.append(seg(pay, idx[h * EPH:(h + 1) * EPH]))
    acc = accs[0]
    for a in accs[1:]:
        acc = acc + a
    a_a, v_flat, d_flat = _unpack(acc)
    return (a_a[:N_NODES],
            v_flat[:N_NODES].reshape(N_NODES, 16, 3),
            d_flat[:N_NODES].reshape(N_NODES, 16, 3, 3))


# speedup vs baseline: 1.0269x; 1.0269x over previous
import functools

import jax
import jax.numpy as jnp
from jax import lax
from jax.experimental import pallas as pl
from jax.experimental.pallas import tpu as pltpu
from jax.experimental.pallas import tpu_sc as plsc

N_NODES = 10000
E = 160000
R0 = 5.0

PW = 128
NC = 2
NS = 16
CHUNK = 128
GRP = 4
NSPLIT = 4
CPT = 80 // NSPLIT
EPT = CPT * CHUNK
EPH = NS * EPT
EP = NSPLIT * EPH
RPT = 632
NPAD = NS * RPT
TCB = 8192


def _leaky(x):
    return jnp.where(x >= 0, x, 0.1 * x)


def _dot(a, b):
    return lax.dot(a, b, precision=lax.Precision.HIGHEST,
                   preferred_element_type=jnp.float32)


def _edge_phi_body(r_ref, w0, b0, wd, w1, b1, w2, b2, w3, b3, wv, wdd,
                   out_ref):
    rt = r_ref[...]
    d2 = jnp.sum(rt * rt, axis=0, keepdims=True)
    x = jnp.sqrt(d2 + 1e-12) * (1.0 / R0)
    mu = lax.broadcasted_iota(jnp.int32, (8, 1), 0).astype(jnp.float32) / 7.0
    enc = jnp.exp(-0.5 * ((x - mu) * 8.0) ** 2)
    h = _dot(w0[...], enc) + b0[...]
    direct = _dot(wd[...], h)
    y = _leaky(_dot(w1[...], h) + b1[...])
    y = _leaky(_dot(w2[...], y) + b2[...])
    y = _dot(w3[...], y) + b3[...]
    rad = direct + y
    gv = _dot(wv[...], rad)
    gd = _dot(wdd[...], rad)
    rs = rt * (7.0 / R0)
    n2 = jnp.sum(rs * rs, axis=0, keepdims=True)
    rh = rs / jnp.sqrt(1.0 + n2)
    r0_, r1_, r2_ = rh[0:1], rh[1:2], rh[2:3]
    p0 = jnp.concatenate(
        [rad, gv * r0_, gv * r1_, gv * r2_, gd * (r0_ * r0_)], axis=0)
    p1 = jnp.concatenate(
        [gd * (r0_ * r1_), gd * (r0_ * r2_), gd * (r1_ * r1_),
         gd * (r1_ * r2_), gd * (r2_ * r2_)], axis=0)
    zpad = jnp.zeros((PW - 80, p0.shape[1]), jnp.float32)
    out_ref[0] = jnp.concatenate([p0, zpad], axis=0).T
    out_ref[1] = jnp.concatenate([p1, zpad], axis=0).T


def _edge_payload(r_t, *ws):
    wspecs = [pl.BlockSpec(w.shape, lambda i: (0, 0)) for w in ws]
    return pl.pallas_call(
        _edge_phi_body,
        grid=(EPH // TCB,),
        in_specs=[pl.BlockSpec((3, TCB), lambda i: (0, i))] + wspecs,
        out_specs=pl.BlockSpec((2, TCB, PW), lambda i: (0, i, 0)),
        out_shape=jax.ShapeDtypeStruct((2, EPH, PW), jnp.float32),
    )(r_t, *ws)


def _unpack_body(*refs):
    accs, (aa_ref, v_ref, d_ref) = refs[:NSPLIT], refs[NSPLIT:]
    o0 = sum(a[0] for a in accs)
    o1 = sum(a[1] for a in accs)
    aa_ref[...] = o0[:, 0:16]
    j = lax.broadcasted_iota(jnp.int32, (48, 48), 0)
    k = lax.broadcasted_iota(jnp.int32, (48, 48), 1)
    pv = ((j % 16) * 3 + j // 16 == k).astype(jnp.float32)
    v_ref[...] = _dot(o0[:, 16:64], pv)
    d6 = jnp.concatenate([o0[:, 64:80], o1], axis=1)
    j = lax.broadcasted_iota(jnp.int32, (96, 144), 0)
    k = lax.broadcasted_iota(jnp.int32, (96, 144), 1)
    r = (k % 9) // 3
    s = k % 3
    lo = jnp.minimum(r, s)
    hi = jnp.maximum(r, s)
    pair = lo * 3 - (lo * (lo - 1)) // 2 + (hi - lo)
    pd = ((j % 16 == k // 9) & (j // 16 == pair)).astype(jnp.float32)
    d_ref[...] = _dot(d6, pd)


def _unpack(*accs):
    nb = NPAD // RPT
    return pl.pallas_call(
        _unpack_body,
        grid=(nb,),
        in_specs=[pl.BlockSpec((2, RPT, 80), lambda i: (0, i, 0))
                  for _ in accs],
        out_specs=[
            pl.BlockSpec((RPT, 16), lambda i: (i, 0)),
            pl.BlockSpec((RPT, 48), lambda i: (i, 0)),
            pl.BlockSpec((RPT, 144), lambda i: (i, 0)),
        ],
        out_shape=[
            jax.ShapeDtypeStruct((NPAD, 16), jnp.float32),
            jax.ShapeDtypeStruct((NPAD, 48), jnp.float32),
            jax.ShapeDtypeStruct((NPAD, 144), jnp.float32),
        ],
    )(*accs)


@functools.cache
def _build_segment_sum_sc():
    mesh = plsc.VectorSubcoreMesh(
        core_axis_name="c", subcore_axis_name="s",
        num_cores=NC, num_subcores=NS)
    return pl.kernel(
        _segment_sum_sc_body,
        out_type=jax.ShapeDtypeStruct((NC, NPAD, 80), jnp.float32),
        mesh=mesh,
        scratch_types=[
            pltpu.VMEM_SHARED((NPAD, 80), jnp.float32),
            pltpu.VMEM((GRP * CHUNK, 80), jnp.float32),
            pltpu.VMEM((GRP, CHUNK), jnp.int32),
            pltpu.VMEM((64, 80), jnp.float32),
            pltpu.SemaphoreType.DMA,
        ],
        compiler_params=pltpu.CompilerParams(use_tc_tiling_on_sc=False),
    )


def _segment_sum_sc_body(pay_hbm, idx_hbm, out_hbm, acc, pbuf, idxbuf, zbuf,
                         sem):
    c = lax.axis_index("c")
    s = lax.axis_index("s")

    def zrow(r, carry):
        for l in range(80 // 16):
            zbuf[r, pl.ds(l * 16, 16)] = jnp.zeros((16,), jnp.float32)
        return carry

    lax.fori_loop(0, 64, zrow, 0)
    for z in range(9):
        pltpu.sync_copy(zbuf, acc.at[pl.ds(s * RPT + z * 64, 64)])
    pltpu.sync_copy(zbuf.at[pl.ds(0, 56)],
                    acc.at[pl.ds(s * RPT + 576, 56)])
    plsc.subcore_barrier()

    def body(g, carry):
        base = s * EPT + g * (GRP * CHUNK)
        for k in range(GRP):
            pltpu.sync_copy(idx_hbm.at[pl.ds(base + k * CHUNK, CHUNK)],
                            idxbuf.at[k])
        pltpu.sync_copy(
            pay_hbm.at[c, pl.ds(base, GRP * CHUNK), pl.ds(0, 80)], pbuf)
        descs = [
            pltpu.async_copy(
                pbuf.at[pl.ds(k * CHUNK, CHUNK)],
                acc.at[idxbuf.at[k]],
                sem, add=True)
            for k in range(GRP)
        ]
        for dsc in descs:
            dsc.wait()
        return carry

    lax.fori_loop(0, CPT // GRP, body, 0)
    plsc.subcore_barrier()
    pltpu.sync_copy(acc.at[pl.ds(s * RPT, RPT)],
                    out_hbm.at[c, pl.ds(s * RPT, RPT)])


def kernel(r_ij, edge_index, W0, b0, Wd, W1, b1, W2, b2, W3, b3, Wv, Wdd):
    src = edge_index[0].astype(jnp.int32)
    r_t = jnp.zeros((3, EP), jnp.float32).at[:, :E].set(r_ij.T)
    idx = jnp.full((EP,), N_NODES, jnp.int32).at[:E].set(src)
    ws = (W0, b0.reshape(-1, 1), Wd, W1, b1.reshape(-1, 1), W2,
          b2.reshape(-1, 1), W3, b3.reshape(-1, 1), Wv, Wdd)
    seg = _build_segment_sum_sc()
    accs = []
    for h in range(NSPLIT):
        pay = _edge_payload(r_t[:, h * EPH:(h + 1) * EPH], *ws)
        accs.append(seg(pay, idx[h * EPH:(h + 1) * EPH]))
    a_a, v_flat, d_flat = _unpack(*accs)
    return (a_a[:N_NODES],
            v_flat[:N_NODES].reshape(N_NODES, 16, 3),
            d_flat[:N_NODES].reshape(N_NODES, 16, 3, 3))
